# Initial kernel scaffold; baseline (speedup 1.0000x reference)
import jax
import jax.numpy as jnp
from jax.experimental import pallas as pl


def kernel(x, edge_index):
    raise NotImplementedError("write your pallas kernel here")


# speedup vs baseline: 3.2810x; 3.2810x over previous
import functools

import jax
import jax.numpy as jnp
from jax import lax
from jax.experimental import pallas as pl
from jax.experimental.pallas import tpu as pltpu
from jax.experimental.pallas import tpu_sc as plsc

N = 10000
E = 320000
D = 128

NC = 2
NS = 16
NW = NC * NS

K = 128
CHUNKS = 80
EPAD = NW * CHUNKS * K
NPAD = N + 112
ZROWS = NPAD // NS
OROWS = 624

_mesh = plsc.VectorSubcoreMesh(core_axis_name="c", subcore_axis_name="s")


@functools.partial(
    pl.kernel,
    mesh=_mesh,
    out_type=jax.ShapeDtypeStruct((NC, N, D), jnp.float32),
    scratch_types=[
        pltpu.VMEM((CHUNKS, K), jnp.int32),
        pltpu.VMEM((CHUNKS, K), jnp.int32),
        pltpu.VMEM((K, D), jnp.float32),
        pltpu.VMEM_SHARED((NPAD, D), jnp.float32),
        pltpu.SemaphoreType.DMA,
    ],
)
def _mp_sc(x_hbm, ei_hbm, out_hbm, dst_v, src_v, rows_v, acc, sem):
    cid = lax.axis_index("c")
    sid = lax.axis_index("s")
    wid = cid * NS + sid

    pltpu.sync_copy(ei_hbm.at[0, wid], dst_v)
    pltpu.sync_copy(ei_hbm.at[1, wid], src_v)

    def _zero_row(r, carry):
        for c in range(D // 16):
            rows_v[r, pl.ds(c * 16, 16)] = jnp.zeros((16,), jnp.float32)
        return carry

    lax.fori_loop(0, K, _zero_row, 0)
    base = sid * ZROWS
    nfull = ZROWS // K
    for j in range(nfull):
        pltpu.sync_copy(rows_v, acc.at[pl.ds(base + j * K, K)])
    rem = ZROWS - nfull * K
    if rem:
        pltpu.sync_copy(rows_v.at[pl.ds(0, rem)],
                        acc.at[pl.ds(base + nfull * K, rem)])
    plsc.subcore_barrier()

    def _body(j, carry):
        pltpu.async_copy(x_hbm.at[src_v.at[j]], rows_v, sem).wait()
        pltpu.sync_copy(rows_v, acc.at[dst_v.at[j]], add=True)
        return carry

    lax.fori_loop(0, CHUNKS, _body, 0)

    plsc.subcore_barrier()

    ob = sid * OROWS
    pltpu.sync_copy(acc.at[pl.ds(ob, OROWS)],
                    out_hbm.at[cid, pl.ds(ob, OROWS)])

    @pl.when(sid == 0)
    def _tail():
        t0 = NS * OROWS
        pltpu.sync_copy(acc.at[pl.ds(t0, N - t0)],
                        out_hbm.at[cid, pl.ds(t0, N - t0)])


def _combine(parts):
    def _add(p_ref, o_ref):
        o_ref[...] = p_ref[0] + p_ref[1]

    return pl.pallas_call(
        _add,
        grid=(10,),
        in_specs=[pl.BlockSpec((2, N // 10, D), lambda i: (0, i, 0))],
        out_specs=pl.BlockSpec((N // 10, D), lambda i: (i, 0)),
        out_shape=jax.ShapeDtypeStruct((N, D), jnp.float32),
    )(parts)


def kernel(x, edge_index):
    pad = EPAD - E
    dst = jnp.concatenate([edge_index[0], jnp.full((pad,), N, jnp.int32)])
    src = jnp.concatenate([edge_index[1], jnp.zeros((pad,), jnp.int32)])
    ei = jnp.stack([dst, src]).reshape(2, NW, CHUNKS, K)
    parts = _mp_sc(x, ei)
    return _combine(parts)


# speedup vs baseline: 3.6339x; 1.1076x over previous
import functools

import jax
import jax.numpy as jnp
from jax import lax
from jax.experimental import pallas as pl
from jax.experimental.pallas import tpu as pltpu
from jax.experimental.pallas import tpu_sc as plsc

N = 10000
E = 320000
D = 128

NC = 2
NS = 16
NW = NC * NS

K = 128
CHUNKS = 80
IPH = 2
IC = CHUNKS // IPH
EPAD = NW * CHUNKS * K
NPAD = N + 112
ZROWS = NPAD // NS
OROWS = 624

_mesh = plsc.VectorSubcoreMesh(core_axis_name="c", subcore_axis_name="s")


@functools.partial(
    pl.kernel,
    mesh=_mesh,
    out_type=jax.ShapeDtypeStruct((NC, N, D), jnp.float32),
    scratch_types=[
        pltpu.VMEM((IC, K), jnp.int32),
        pltpu.VMEM((IC, K), jnp.int32),
        pltpu.VMEM((K, D), jnp.float32),
        pltpu.VMEM((K, D), jnp.float32),
        pltpu.VMEM_SHARED((NPAD, D), jnp.float32),
        pltpu.SemaphoreType.DMA,
        pltpu.SemaphoreType.DMA,
        pltpu.SemaphoreType.DMA,
    ],
)
def _mp_sc(x_hbm, ei_hbm, out_hbm, dst_v, src_v, rows_a, rows_b, acc,
           sem_a, sem_b, sem_i):
    cid = lax.axis_index("c")
    sid = lax.axis_index("s")
    wid = cid * NS + sid

    cp_d = pltpu.async_copy(ei_hbm.at[0, wid, 0], dst_v, sem_i)
    cp_s = pltpu.async_copy(ei_hbm.at[1, wid, 0], src_v, sem_i)

    def _zero_row(r, carry):
        for c in range(D // 16):
            rows_a[r, pl.ds(c * 16, 16)] = jnp.zeros((16,), jnp.float32)
        return carry

    lax.fori_loop(0, K, _zero_row, 0)
    base = sid * ZROWS
    nfull = ZROWS // K
    for j in range(nfull):
        pltpu.sync_copy(rows_a, acc.at[pl.ds(base + j * K, K)])
    rem = ZROWS - nfull * K
    if rem:
        pltpu.sync_copy(rows_a.at[pl.ds(0, rem)],
                        acc.at[pl.ds(base + nfull * K, rem)])
    cp_d.wait()
    cp_s.wait()
    plsc.subcore_barrier()

    def _group(g, carry):
        c0 = 2 * g
        pltpu.async_copy(x_hbm.at[src_v.at[c0 + 1]], rows_b, sem_b)
        pltpu.make_async_copy(x_hbm.at[src_v.at[c0]], rows_a, sem_a).wait()
        pltpu.sync_copy(rows_a, acc.at[dst_v.at[c0]], add=True)
        pltpu.async_copy(x_hbm.at[src_v.at[c0 + 2]], rows_a, sem_a)
        pltpu.make_async_copy(x_hbm.at[src_v.at[c0 + 1]], rows_b, sem_b).wait()
        pltpu.sync_copy(rows_b, acc.at[dst_v.at[c0 + 1]], add=True)
        return carry

    for ph in range(IPH):
        if ph > 0:
            cp_d = pltpu.async_copy(ei_hbm.at[0, wid, ph], dst_v, sem_i)
            cp_s = pltpu.async_copy(ei_hbm.at[1, wid, ph], src_v, sem_i)
            cp_d.wait()
            cp_s.wait()
        pltpu.async_copy(x_hbm.at[src_v.at[0]], rows_a, sem_a)
        lax.fori_loop(0, IC // 2 - 1, _group, 0)
        cl = IC - 2
        pltpu.async_copy(x_hbm.at[src_v.at[cl + 1]], rows_b, sem_b)
        pltpu.make_async_copy(x_hbm.at[src_v.at[cl]], rows_a, sem_a).wait()
        pltpu.sync_copy(rows_a, acc.at[dst_v.at[cl]], add=True)
        pltpu.make_async_copy(x_hbm.at[src_v.at[cl + 1]], rows_b, sem_b).wait()
        pltpu.sync_copy(rows_b, acc.at[dst_v.at[cl + 1]], add=True)

    plsc.subcore_barrier()

    ob = sid * OROWS
    pltpu.sync_copy(acc.at[pl.ds(ob, OROWS)],
                    out_hbm.at[cid, pl.ds(ob, OROWS)])

    @pl.when(sid == 0)
    def _tail():
        t0 = NS * OROWS
        pltpu.sync_copy(acc.at[pl.ds(t0, N - t0)],
                        out_hbm.at[cid, pl.ds(t0, N - t0)])


def _combine(parts):
    def _add(p_ref, o_ref):
        o_ref[...] = p_ref[0] + p_ref[1]

    return pl.pallas_call(
        _add,
        grid=(10,),
        in_specs=[pl.BlockSpec((2, N // 10, D), lambda i: (0, i, 0))],
        out_specs=pl.BlockSpec((N // 10, D), lambda i: (i, 0)),
        out_shape=jax.ShapeDtypeStruct((N, D), jnp.float32),
    )(parts)


def kernel(x, edge_index):
    pad = EPAD - E
    dst = jnp.concatenate([edge_index[0], jnp.full((pad,), N, jnp.int32)])
    src = jnp.concatenate([edge_index[1], jnp.zeros((pad,), jnp.int32)])
    ei = jnp.stack([dst, src]).reshape(2, NW, IPH, IC, K)
    parts = _mp_sc(x, ei)
    return _combine(parts)
